# baseline (speedup 1.0000x reference)
import jax
import jax.numpy as jnp
from jax.experimental import pallas as pl

N = 10000
D = 128
BM = 400


def _zi_kernel(a_ref, z1_ref, b_ref, z2_ref, zi_ref):
    zi_ref[...] = a_ref[...] * z1_ref[...] + b_ref[...] * z2_ref[...]


def _matmul_fuse_kernel(alpha_ref, adj_ref, zi_ref, out_ref):
    i = pl.program_id(0)
    alpha = alpha_ref[0, 0]
    zi = zi_ref[...]
    zl = jnp.dot(adj_ref[...], zi, preferred_element_type=jnp.float32)
    zi_rows = zi_ref[pl.ds(i * BM, BM), :]
    out_ref[...] = alpha * zl + (1.0 - alpha) * zi_rows


def kernel(z1, z2, adj, a, b, alpha):
    zi = pl.pallas_call(
        _zi_kernel,
        grid=(10,),
        in_specs=[pl.BlockSpec((N // 10, D), lambda i: (i, 0))] * 4,
        out_specs=pl.BlockSpec((N // 10, D), lambda i: (i, 0)),
        out_shape=jax.ShapeDtypeStruct((N, D), jnp.float32),
    )(a, z1, b, z2)

    alpha_arr = jnp.reshape(alpha.astype(jnp.float32), (1, 1))
    out = pl.pallas_call(
        _matmul_fuse_kernel,
        grid=(N // BM,),
        in_specs=[
            pl.BlockSpec((1, 1), lambda i: (0, 0)),
            pl.BlockSpec((BM, N), lambda i: (i, 0)),
            pl.BlockSpec((N, D), lambda i: (0, 0)),
        ],
        out_specs=pl.BlockSpec((BM, D), lambda i: (i, 0)),
        out_shape=jax.ShapeDtypeStruct((N, D), jnp.float32),
    )(alpha_arr, adj, zi)
    return out


# speedup vs baseline: 1.0482x; 1.0482x over previous
import jax
import jax.numpy as jnp
from jax.experimental import pallas as pl
from jax.experimental.pallas import tpu as pltpu

N = 10000
D = 128
BM = 200


def _fused_kernel(alpha_ref, z1_ref, z2_ref, a_ref, b_ref, adj_ref, out_ref,
                  zi_ref):
    i = pl.program_id(0)

    @pl.when(i == 0)
    def _():
        zi_ref[...] = a_ref[...] * z1_ref[...] + b_ref[...] * z2_ref[...]

    alpha = alpha_ref[0, 0]
    zi = zi_ref[...]
    zl = jnp.dot(adj_ref[...], zi, preferred_element_type=jnp.float32)
    zi_rows = zi_ref[pl.ds(i * BM, BM), :]
    out_ref[...] = alpha * zl + (1.0 - alpha) * zi_rows


def kernel(z1, z2, adj, a, b, alpha):
    alpha_arr = jnp.reshape(alpha.astype(jnp.float32), (1, 1))
    full = pl.BlockSpec((N, D), lambda i: (0, 0))
    return pl.pallas_call(
        _fused_kernel,
        grid=(N // BM,),
        in_specs=[
            pl.BlockSpec((1, 1), lambda i: (0, 0)),
            full, full, full, full,
            pl.BlockSpec((BM, N), lambda i: (i, 0)),
        ],
        out_specs=pl.BlockSpec((BM, D), lambda i: (i, 0)),
        out_shape=jax.ShapeDtypeStruct((N, D), jnp.float32),
        scratch_shapes=[pltpu.VMEM((N, D), jnp.float32)],
    )(alpha_arr, z1, z2, a, b, adj)


# speedup vs baseline: 1.0545x; 1.0060x over previous
import jax
import jax.numpy as jnp
from jax.experimental import pallas as pl
from jax.experimental.pallas import tpu as pltpu

N = 10000
D = 128
BM = 400


def _fused_kernel(alpha_ref, z1_ref, z2_ref, a_ref, b_ref, adj_ref, out_ref,
                  zi_ref):
    i = pl.program_id(0)

    @pl.when(i == 0)
    def _():
        zi_ref[...] = a_ref[...] * z1_ref[...] + b_ref[...] * z2_ref[...]

    alpha = alpha_ref[0, 0]
    zi = zi_ref[...]
    zl = jnp.dot(adj_ref[...], zi, preferred_element_type=jnp.float32)
    zi_rows = zi_ref[pl.ds(i * BM, BM), :]
    out_ref[...] = alpha * zl + (1.0 - alpha) * zi_rows


def kernel(z1, z2, adj, a, b, alpha):
    alpha_arr = jnp.reshape(alpha.astype(jnp.float32), (1, 1))
    full = pl.BlockSpec((N, D), lambda i: (0, 0))
    return pl.pallas_call(
        _fused_kernel,
        grid=(N // BM,),
        in_specs=[
            pl.BlockSpec((1, 1), lambda i: (0, 0)),
            full, full, full, full,
            pl.BlockSpec((BM, N), lambda i: (i, 0)),
        ],
        out_specs=pl.BlockSpec((BM, D), lambda i: (i, 0)),
        out_shape=jax.ShapeDtypeStruct((N, D), jnp.float32),
        scratch_shapes=[pltpu.VMEM((N, D), jnp.float32)],
    )(alpha_arr, z1, z2, a, b, adj)
